# Initial kernel scaffold; baseline (speedup 1.0000x reference)
import jax
import jax.numpy as jnp
from jax.experimental import pallas as pl


def kernel(node_embeddings, node_to_graph_map, num_graphs, W_s1, W_s2, W_t1, W_t2):
    raise NotImplementedError("write your pallas kernel here")


# speedup vs baseline: 15.2245x; 15.2245x over previous
import functools

import jax
import jax.numpy as jnp
from jax import lax
from jax.experimental import pallas as pl
from jax.experimental.pallas import tpu as pltpu
from jax.experimental.pallas import tpu_sc as plsc

G = 128
H = 4
NEG = -1e30
EPS = 1e-7
TILE = 2000


def _stats_body(x_ref, seg_ref, ws1_ref, ws2_ref, wt1_ref, wt2_ref,
                pg_ref, m_ref, s_ref, n_ref):
    T, D = x_ref.shape
    HS = D // H
    i = pl.program_id(0)
    nt = pl.num_programs(0)

    @pl.when(i == 0)
    def _():
        m_ref[...] = jnp.full((H, G), NEG, jnp.float32)
        s_ref[...] = jnp.zeros((H, G), jnp.float32)
        n_ref[...] = jnp.zeros((D, G), jnp.float32)

    x = x_ref[...]
    seg = seg_ref[...]
    gids = lax.broadcasted_iota(jnp.int32, (T, G), 1)
    mask = seg == gids
    mask_f = mask.astype(jnp.float32)

    dn = (((1,), (0,)), ((), ()))
    h1 = jnp.maximum(
        lax.dot_general(x, ws1_ref[...], dn, preferred_element_type=jnp.float32), 0.0)
    sc = lax.dot_general(h1, ws2_ref[...], dn, preferred_element_type=jnp.float32)
    v1 = jnp.maximum(
        lax.dot_general(x, wt1_ref[...], dn, preferred_element_type=jnp.float32), 0.0)
    vals = lax.dot_general(v1, wt2_ref[...], dn, preferred_element_type=jnp.float32)

    tm = jnp.concatenate(
        [jnp.max(jnp.where(mask, sc[:, h:h + 1], NEG), axis=0, keepdims=True)
         for h in range(H)], axis=0)
    m_old = m_ref[...]
    m_new = jnp.maximum(m_old, tm)
    m_ref[...] = m_new
    scale = jnp.exp(m_old - m_new)

    gmax = lax.dot_general(mask_f, m_new, (((1,), (1,)), ((), ())),
                           preferred_element_type=jnp.float32)
    esc = jnp.exp(sc - gmax)

    s_ref[...] = s_ref[...] * scale + lax.dot_general(
        esc, mask_f, (((0,), (0,)), ((), ())),
        preferred_element_type=jnp.float32)

    esc_exp = jnp.concatenate(
        [jnp.broadcast_to(esc[:, h:h + 1], (T, HS)) for h in range(H)], axis=1)
    weighted = vals * esc_exp
    scale_exp = jnp.concatenate(
        [jnp.broadcast_to(scale[h:h + 1, :], (HS, G)) for h in range(H)], axis=0)
    n_ref[...] = n_ref[...] * scale_exp + lax.dot_general(
        weighted, mask_f, (((0,), (0,)), ((), ())),
        preferred_element_type=jnp.float32)

    @pl.when(i == nt - 1)
    def _():
        s = s_ref[...]
        s_exp = jnp.concatenate(
            [jnp.broadcast_to(s[h:h + 1, :], (HS, G)) for h in range(H)], axis=0)
        pg_ref[...] = n_ref[...] / (s_exp + EPS)


def _per_graph_table(x, seg_col, ws1, ws2, wt1, wt2):
    V, D = x.shape
    nt = V // TILE
    return pl.pallas_call(
        _stats_body,
        grid=(nt,),
        in_specs=[
            pl.BlockSpec((TILE, D), lambda i: (i, 0)),
            pl.BlockSpec((TILE, 1), lambda i: (i, 0)),
            pl.BlockSpec((D, D), lambda i: (0, 0)),
            pl.BlockSpec((D, H), lambda i: (0, 0)),
            pl.BlockSpec((D, D), lambda i: (0, 0)),
            pl.BlockSpec((D, D), lambda i: (0, 0)),
        ],
        out_specs=pl.BlockSpec((D, G), lambda i: (0, 0)),
        out_shape=jax.ShapeDtypeStruct((D, G), jnp.float32),
        scratch_shapes=[
            pltpu.VMEM((H, G), jnp.float32),
            pltpu.VMEM((H, G), jnp.float32),
            pltpu.VMEM((D, G), jnp.float32),
        ],
    )(x, seg_col, ws1, ws2, wt1, wt2)


def _gather_rows(table, idx3d):
    nw, ch, chunk = idx3d.shape
    d = table.shape[1]
    vp = nw * ch * chunk
    rows_pw = ch * chunk
    info = plsc.get_sparse_core_info()
    nc = info.num_cores
    mesh = plsc.VectorSubcoreMesh(core_axis_name="c", subcore_axis_name="s")

    @functools.partial(
        pl.kernel, mesh=mesh,
        out_type=jax.ShapeDtypeStruct((vp, d), jnp.float32),
        scratch_types=[
            pltpu.VMEM((ch, chunk), jnp.int32),
            pltpu.VMEM((chunk, d), jnp.float32),
            pltpu.SemaphoreType.DMA,
        ],
    )
    def k(table_hbm, idx_hbm, out_hbm, idx_v, buf_v, sem):
        wid = lax.axis_index("s") * nc + lax.axis_index("c")
        base = wid * rows_pw
        pltpu.sync_copy(idx_hbm.at[wid], idx_v)

        def body(j, carry):
            pltpu.async_copy(table_hbm.at[idx_v.at[j]], buf_v, sem).wait()
            pltpu.sync_copy(buf_v, out_hbm.at[pl.ds(base + j * chunk, chunk)])
            return carry

        lax.fori_loop(0, ch, body, 0)

    return k(table, idx3d)


def kernel(node_embeddings, node_to_graph_map, num_graphs, W_s1, W_s2, W_t1, W_t2):
    x = node_embeddings
    V, D = x.shape
    seg = node_to_graph_map.astype(jnp.int32)

    pg_t = _per_graph_table(x, seg.reshape(V, 1), W_s1, W_s2, W_t1, W_t2)
    per_graph = pg_t.T

    nw, chunk = 32, 128
    rows_pw = -(-V // (nw * chunk)) * chunk
    vp = nw * rows_pw
    idx_pad = jnp.concatenate(
        [seg, jnp.zeros((vp - V,), jnp.int32)]).reshape(nw, rows_pw // chunk, chunk)
    out_pad = _gather_rows(per_graph, idx_pad)
    return out_pad[:V]


# speedup vs baseline: 21.2968x; 1.3989x over previous
import functools

import jax
import jax.numpy as jnp
from jax import lax
from jax.experimental import pallas as pl
from jax.experimental.pallas import tpu as pltpu
from jax.experimental.pallas import tpu_sc as plsc

G = 128
H = 4
NEG = -1e30
EPS = 1e-7
TILE = 2000

NW = 32
CHUNK = 128
NBUF = 4


def _stats_body(x_ref, seg_ref, ws1_ref, ws2_ref, wt1_ref, wt2_ref,
                pg_ref, m_ref, s_ref, n_ref):
    T, D = x_ref.shape
    HS = D // H
    i = pl.program_id(0)
    nt = pl.num_programs(0)

    @pl.when(i == 0)
    def _():
        m_ref[...] = jnp.full((G, H), NEG, jnp.float32)
        s_ref[...] = jnp.zeros((G, H), jnp.float32)
        n_ref[...] = jnp.zeros((G, D), jnp.float32)

    x = x_ref[...]
    seg_row = seg_ref[0]
    mask = lax.broadcasted_iota(jnp.int32, (G, T), 0) == seg_row
    mask_f = mask.astype(jnp.float32)

    dn = (((1,), (0,)), ((), ()))
    h1 = jnp.maximum(
        lax.dot_general(x, ws1_ref[...], dn, preferred_element_type=jnp.float32), 0.0)
    sc = lax.dot_general(h1, ws2_ref[...], dn, preferred_element_type=jnp.float32)
    sc_t = lax.dot_general(ws2_ref[...], h1, (((0,), (1,)), ((), ())),
                           preferred_element_type=jnp.float32)
    v1 = jnp.maximum(
        lax.dot_general(x, wt1_ref[...], dn, preferred_element_type=jnp.float32), 0.0)
    vals = lax.dot_general(v1, wt2_ref[...], dn, preferred_element_type=jnp.float32)

    tm = jnp.concatenate(
        [jnp.max(jnp.where(mask, sc_t[h:h + 1, :], NEG), axis=1, keepdims=True)
         for h in range(H)], axis=1)
    m_old = m_ref[...]
    m_new = jnp.maximum(m_old, tm)
    m_ref[...] = m_new
    scale = jnp.exp(m_old - m_new)

    gmax = lax.dot_general(mask_f, m_new, (((0,), (0,)), ((), ())),
                           preferred_element_type=jnp.float32)
    esc = jnp.exp(sc - gmax)

    s_ref[...] = s_ref[...] * scale + lax.dot_general(
        mask_f, esc, (((1,), (0,)), ((), ())),
        preferred_element_type=jnp.float32)

    esc_exp = jnp.concatenate(
        [jnp.broadcast_to(esc[:, h:h + 1], (T, HS)) for h in range(H)], axis=1)
    weighted = vals * esc_exp
    scale_exp = jnp.concatenate(
        [jnp.broadcast_to(scale[:, h:h + 1], (G, HS)) for h in range(H)], axis=1)
    n_ref[...] = n_ref[...] * scale_exp + lax.dot_general(
        mask_f, weighted, (((1,), (0,)), ((), ())),
        preferred_element_type=jnp.float32)

    @pl.when(i == nt - 1)
    def _():
        s = s_ref[...]
        s_exp = jnp.concatenate(
            [jnp.broadcast_to(s[:, h:h + 1], (G, HS)) for h in range(H)], axis=1)
        pg_ref[...] = n_ref[...] / (s_exp + EPS)


def _per_graph_table(x, seg3d, ws1, ws2, wt1, wt2):
    V, D = x.shape
    nt = V // TILE
    return pl.pallas_call(
        _stats_body,
        grid=(nt,),
        in_specs=[
            pl.BlockSpec((TILE, D), lambda i: (i, 0)),
            pl.BlockSpec((1, 1, TILE), lambda i: (i, 0, 0)),
            pl.BlockSpec((D, D), lambda i: (0, 0)),
            pl.BlockSpec((D, H), lambda i: (0, 0)),
            pl.BlockSpec((D, D), lambda i: (0, 0)),
            pl.BlockSpec((D, D), lambda i: (0, 0)),
        ],
        out_specs=pl.BlockSpec((G, D), lambda i: (0, 0)),
        out_shape=jax.ShapeDtypeStruct((G, D), jnp.float32),
        scratch_shapes=[
            pltpu.VMEM((G, H), jnp.float32),
            pltpu.VMEM((G, H), jnp.float32),
            pltpu.VMEM((G, D), jnp.float32),
        ],
    )(x, seg3d, ws1, ws2, wt1, wt2)


def _gather_rows(table, idx3d, offs, rows_pw, v):
    nw, ch, chunk = idx3d.shape
    d = table.shape[1]
    info = plsc.get_sparse_core_info()
    nc = info.num_cores
    mesh = plsc.VectorSubcoreMesh(core_axis_name="c", subcore_axis_name="s")

    @functools.partial(
        pl.kernel, mesh=mesh,
        out_type=jax.ShapeDtypeStruct((v, d), jnp.float32),
        scratch_types=[
            pltpu.VMEM((ch, chunk), jnp.int32),
            pltpu.VMEM((NBUF, chunk, d), jnp.float32),
            pltpu.SemaphoreType.DMA((NBUF,)),
            pltpu.SemaphoreType.DMA((NBUF,)),
        ],
    )
    def k(table_hbm, idx_hbm, out_hbm, idx_v, buf_v, gsem, wsem):
        wid = lax.axis_index("s") * nc + lax.axis_index("c")
        base = jnp.minimum(wid * rows_pw, v - rows_pw)
        pltpu.sync_copy(idx_hbm.at[wid], idx_v)

        gh = {}
        wh = {}
        for j in range(min(NBUF, ch)):
            gh[j] = pltpu.async_copy(
                table_hbm.at[idx_v.at[j]], buf_v.at[j % NBUF], gsem.at[j % NBUF])
        for j in range(ch):
            b = j % NBUF
            gh[j].wait()
            wh[j] = pltpu.async_copy(
                buf_v.at[b], out_hbm.at[pl.ds(base + offs[j], chunk)], wsem.at[b])
            if j + NBUF < ch:
                wh[j].wait()
                gh[j + NBUF] = pltpu.async_copy(
                    table_hbm.at[idx_v.at[j + NBUF]], buf_v.at[b], gsem.at[b])
        for j in range(max(ch - NBUF, 0), ch):
            wh[j].wait()

    return k(table, idx3d)


def kernel(node_embeddings, node_to_graph_map, num_graphs, W_s1, W_s2, W_t1, W_t2):
    x = node_embeddings
    V, D = x.shape
    seg = node_to_graph_map.astype(jnp.int32)
    nt = V // TILE

    pg = _per_graph_table(x, seg.reshape(nt, 1, TILE), W_s1, W_s2, W_t1, W_t2)

    rows_pw = -(-V // (NW * 8)) * 8
    ch = -(-rows_pw // CHUNK)
    offs = [min(j * CHUNK, rows_pw - CHUNK) for j in range(ch)]
    bases = [min(w * rows_pw, V - rows_pw) for w in range(NW)]
    gidx = (jnp.asarray(bases, dtype=jnp.int32)[:, None, None]
            + jnp.asarray(offs, dtype=jnp.int32)[None, :, None]
            + jnp.arange(CHUNK, dtype=jnp.int32)[None, None, :])
    idx3d = jnp.take(seg, gidx, axis=0)
    return _gather_rows(pg, idx3d, offs, rows_pw, V)


# speedup vs baseline: 22.2039x; 1.0426x over previous
import functools

import jax
import jax.numpy as jnp
from jax import lax
from jax.experimental import pallas as pl
from jax.experimental.pallas import tpu as pltpu
from jax.experimental.pallas import tpu_sc as plsc

G = 128
H = 4
NEG = -1e30
EPS = 1e-7
TILE = 2000

NW = 32
CHUNK = 128
NBUF = 6


def _stats_body(x_ref, seg_ref, ws1_ref, ws2_ref, wt1_ref, wt2_ref,
                pg_ref, m_ref, s_ref, n_ref):
    T, D = x_ref.shape
    HS = D // H
    i = pl.program_id(0)
    nt = pl.num_programs(0)

    @pl.when(i == 0)
    def _():
        m_ref[...] = jnp.full((G, 1), NEG, jnp.float32)
        s_ref[...] = jnp.zeros((G, H), jnp.float32)
        n_ref[...] = jnp.zeros((G, D), jnp.float32)

    x = x_ref[...]
    seg_row = seg_ref[0]
    mask = lax.broadcasted_iota(jnp.int32, (G, T), 0) == seg_row
    mask_f = mask.astype(jnp.float32)

    dn = (((1,), (0,)), ((), ()))
    h1 = jnp.maximum(
        lax.dot_general(x, ws1_ref[...], dn, preferred_element_type=jnp.float32), 0.0)
    sc = lax.dot_general(h1, ws2_ref[...], dn, preferred_element_type=jnp.float32)
    sc_t = lax.dot_general(ws2_ref[...], h1, (((0,), (1,)), ((), ())),
                           preferred_element_type=jnp.float32)
    v1 = jnp.maximum(
        lax.dot_general(x, wt1_ref[...], dn, preferred_element_type=jnp.float32), 0.0)
    vals = lax.dot_general(v1, wt2_ref[...], dn, preferred_element_type=jnp.float32)

    rowmax = jnp.max(sc_t, axis=0, keepdims=True)
    tm = jnp.max(jnp.where(mask, rowmax, NEG), axis=1, keepdims=True)
    m_old = m_ref[...]
    m_new = jnp.maximum(m_old, tm)
    m_ref[...] = m_new
    scale = jnp.exp(m_old - m_new)

    gmax = lax.dot_general(mask_f, m_new, (((0,), (0,)), ((), ())),
                           preferred_element_type=jnp.float32)
    esc = jnp.exp(sc - gmax)

    s_ref[...] = s_ref[...] * scale + lax.dot_general(
        mask_f, esc, (((1,), (0,)), ((), ())),
        preferred_element_type=jnp.float32)

    esc_exp = jnp.concatenate(
        [jnp.broadcast_to(esc[:, h:h + 1], (T, HS)) for h in range(H)], axis=1)
    weighted = vals * esc_exp
    n_ref[...] = n_ref[...] * scale + lax.dot_general(
        mask_f, weighted, (((1,), (0,)), ((), ())),
        preferred_element_type=jnp.float32)

    @pl.when(i == nt - 1)
    def _():
        s = s_ref[...]
        s_exp = jnp.concatenate(
            [jnp.broadcast_to(s[:, h:h + 1], (G, HS)) for h in range(H)], axis=1)
        pg_ref[...] = n_ref[...] / (s_exp + EPS)


def _per_graph_table(x, seg3d, ws1, ws2, wt1, wt2):
    V, D = x.shape
    nt = V // TILE
    return pl.pallas_call(
        _stats_body,
        grid=(nt,),
        in_specs=[
            pl.BlockSpec((TILE, D), lambda i: (i, 0)),
            pl.BlockSpec((1, 1, TILE), lambda i: (i, 0, 0)),
            pl.BlockSpec((D, D), lambda i: (0, 0)),
            pl.BlockSpec((D, H), lambda i: (0, 0)),
            pl.BlockSpec((D, D), lambda i: (0, 0)),
            pl.BlockSpec((D, D), lambda i: (0, 0)),
        ],
        out_specs=pl.BlockSpec((G, D), lambda i: (0, 0)),
        out_shape=jax.ShapeDtypeStruct((G, D), jnp.float32),
        scratch_shapes=[
            pltpu.VMEM((G, 1), jnp.float32),
            pltpu.VMEM((G, H), jnp.float32),
            pltpu.VMEM((G, D), jnp.float32),
        ],
    )(x, seg3d, ws1, ws2, wt1, wt2)


def _gather_rows(rep_table, seg, offs, rows_pw, v):
    nw, g, d = rep_table.shape
    ch = len(offs)
    info = plsc.get_sparse_core_info()
    nc = info.num_cores
    mesh = plsc.VectorSubcoreMesh(core_axis_name="c", subcore_axis_name="s")

    @functools.partial(
        pl.kernel, mesh=mesh,
        out_type=jax.ShapeDtypeStruct((v, d), jnp.float32),
        scratch_types=[
            pltpu.VMEM((ch * CHUNK,), jnp.int32),
            pltpu.VMEM((NBUF, CHUNK, d), jnp.float32),
            pltpu.SemaphoreType.DMA((NBUF,)),
            pltpu.SemaphoreType.DMA((NBUF,)),
        ],
    )
    def k(table_hbm, seg_hbm, out_hbm, idx_v, buf_v, gsem, wsem):
        wid = lax.axis_index("s") * nc + lax.axis_index("c")
        base = jnp.minimum(wid * rows_pw, v - rows_pw)
        tbl = table_hbm.at[wid]
        pltpu.sync_copy(seg_hbm.at[pl.ds(base, (ch - 1) * CHUNK)],
                        idx_v.at[pl.ds(0, (ch - 1) * CHUNK)])
        pltpu.sync_copy(seg_hbm.at[pl.ds(base + offs[-1], CHUNK)],
                        idx_v.at[pl.ds((ch - 1) * CHUNK, CHUNK)])

        gh = {}
        wh = {}
        for j in range(min(NBUF, ch)):
            gh[j] = pltpu.async_copy(
                tbl.at[idx_v.at[pl.ds(j * CHUNK, CHUNK)]],
                buf_v.at[j % NBUF], gsem.at[j % NBUF])
        for j in range(ch):
            b = j % NBUF
            gh[j].wait()
            wh[j] = pltpu.async_copy(
                buf_v.at[b], out_hbm.at[pl.ds(base + offs[j], CHUNK)], wsem.at[b])
            if j + NBUF < ch:
                wh[j].wait()
                gh[j + NBUF] = pltpu.async_copy(
                    tbl.at[idx_v.at[pl.ds((j + NBUF) * CHUNK, CHUNK)]],
                    buf_v.at[b], gsem.at[b])
        for j in range(max(ch - NBUF, 0), ch):
            wh[j].wait()

    return k(rep_table, seg)


def kernel(node_embeddings, node_to_graph_map, num_graphs, W_s1, W_s2, W_t1, W_t2):
    x = node_embeddings
    V, D = x.shape
    seg = node_to_graph_map.astype(jnp.int32)
    nt = V // TILE

    pg = _per_graph_table(x, seg.reshape(nt, 1, TILE), W_s1, W_s2, W_t1, W_t2)
    rep = jnp.broadcast_to(pg[None], (NW, G, D))

    rows_pw = -(-V // (NW * 8)) * 8
    ch = -(-rows_pw // CHUNK)
    offs = [min(j * CHUNK, rows_pw - CHUNK) for j in range(ch)]
    return _gather_rows(rep, seg, offs, rows_pw, V)
